# baseline (speedup 1.0000x reference)
import functools

import jax
import jax.numpy as jnp
from jax import lax
from jax.experimental import pallas as pl
from jax.experimental.pallas import tpu as pltpu
from jax.experimental.pallas import tpu_sc as plsc

B = 4096
S = 200
D = 64
VOCAB = 1000000
DP = 128
NC, NS = 2, 16
NW = NC * NS
LANES = 16
DG = D // 8
BL = 128
SCH = 8
BH = 64
NSTEP = (S // SCH) * 2


PITCH = 64
BHP = 65


def _body(x_ref, table_ref, pos_ref, out_ref,
          idx_a, idx_b, buf_a, buf_b, obuf_v, pos_v, gsem_a, gsem_b):
    wid = lax.axis_index("s") * NC + lax.axis_index("c")
    pltpu.sync_copy(pos_ref, pos_v)
    iota = lax.iota(jnp.int32, LANES)
    sg = [(iota + dd * LANES) // 8 for dd in range(D // LANES)]
    sd = [(iota + dd * LANES) % 8 for dd in range(D // LANES)]
    zz = iota * 0

    def gather_issue(t, idx_v, buf_v, gsem):
        bb0 = wid * 8 + (t % 2) * 4
        pltpu.sync_copy(x_ref.at[t // 2, pl.ds(bb0, 4)], idx_v)
        for j in range(4):
            pltpu.async_copy(
                table_ref.at[idx_v.at[j]],
                buf_v.at[pl.ds(j * 128, 128)],
                gsem,
            )

    def gather_wait(idx_v, buf_v, gsem):
        for j in range(4):
            pltpu.make_async_copy(
                table_ref.at[idx_v.at[j]],
                buf_v.at[pl.ds(j * 128, 128)],
                gsem,
            ).wait()

    def compute_store(t, buf_v):
        s0 = (t // 2) * SCH
        half = t % 2

        def seq(sj, carry2):
            pvecs = [pos_v[s0 + sj, pl.ds(dd * LANES, LANES)]
                     for dd in range(D // LANES)]
            ob = obuf_v.at[sj]
            for b16 in range(BH // LANES):
                for bq in range(4):
                    rows = [b16 * 128 + sj * LANES + bq * 4 + bi
                            for bi in range(4)]
                    vals = [
                        buf_v[r, pl.ds(dd * LANES, LANES)]
                        for r in rows
                        for dd in range(D // LANES)
                    ]
                    sums = [
                        vals[k * 4 + dd] + pvecs[dd]
                        for k in range(4)
                        for dd in range(D // LANES)
                    ]
                    for k in range(4):
                        bcol = b16 * 16 + bq * 4 + k
                        for dd in range(D // LANES):
                            plsc.store_scatter(
                                ob,
                                [sg[dd], zz, sd[dd], zz + bcol],
                                sums[k * 4 + dd],
                            )
            return carry2

        lax.fori_loop(0, SCH, seq, 0)
        pltpu.sync_copy(
            obuf_v.at[:, :, :, :, pl.ds(0, BH)],
            out_ref.at[
                pl.ds(s0, SCH), :, pl.ds(wid, 1), :, pl.ds(half * BH, BH)
            ],
        )

    gather_issue(0, idx_a, buf_a, gsem_a)
    gather_issue(1, idx_b, buf_b, gsem_b)

    def pair(k, carry):
        t0 = 2 * k
        gather_wait(idx_a, buf_a, gsem_a)
        compute_store(t0, buf_a)

        @pl.when(t0 + 2 < NSTEP)
        def _():
            gather_issue(t0 + 2, idx_a, buf_a, gsem_a)

        gather_wait(idx_b, buf_b, gsem_b)
        compute_store(t0 + 1, buf_b)

        @pl.when(t0 + 3 < NSTEP)
        def _():
            gather_issue(t0 + 3, idx_b, buf_b, gsem_b)

        return carry

    lax.fori_loop(0, NSTEP // 2, pair, 0)


@functools.lru_cache(maxsize=1)
def _make_gather_add():
    mesh = plsc.VectorSubcoreMesh(
        core_axis_name="c", subcore_axis_name="s", num_cores=NC, num_subcores=NS
    )
    return pl.kernel(
        _body,
        out_type=jax.ShapeDtypeStruct((S, DG, B // BL, 8, BL), jnp.float32),
        mesh=mesh,
        scratch_types=[
            pltpu.VMEM((4, 128), jnp.int32),
            pltpu.VMEM((4, 128), jnp.int32),
            pltpu.VMEM((BH * SCH, PITCH), jnp.float32),
            pltpu.VMEM((BH * SCH, PITCH), jnp.float32),
            pltpu.VMEM((SCH, DG, 1, 8, BHP), jnp.float32),
            pltpu.VMEM((S, D), jnp.float32),
            pltpu.SemaphoreType.DMA,
            pltpu.SemaphoreType.DMA,
        ],
        compiler_params=pltpu.CompilerParams(
            use_tc_tiling_on_sc=False, needs_layout_passes=False
        ),
    )


def kernel(x, table, pos_embed):
    x2 = x.astype(jnp.int32) * 2
    xb = (
        x2.reshape(B // 16, 16, S // SCH, SCH)
        .transpose(2, 0, 3, 1)
        .reshape(S // SCH, B // 16, 128)
    )
    table_w = jnp.pad(table, ((0, 0), (0, DP - D))).reshape(2 * VOCAB, D)
    pos2d = pos_embed.reshape(S, D).astype(jnp.float32)
    out5 = _make_gather_add()(xb, table_w, pos2d)
    return out5.transpose(2, 4, 0, 1, 3).reshape(B, S, D)


# speedup vs baseline: 1.0407x; 1.0407x over previous
import functools

import jax
import jax.numpy as jnp
from jax import lax
from jax.experimental import pallas as pl
from jax.experimental.pallas import tpu as pltpu
from jax.experimental.pallas import tpu_sc as plsc

B = 4096
S = 200
D = 64
VOCAB = 1000000
DP = 128
NC, NS = 2, 16
NW = NC * NS
LANES = 16
DG = D // 8
BL = 128
SCH = 8
BH = 64
NSTEP = (S // SCH) * 2


PITCH = 64
BHP = 65


def _body(x_ref, table_ref, pos_ref, out_ref, idx_v, buf_v, obuf_v, pos_v, sem):
    wid = lax.axis_index("s") * NC + lax.axis_index("c")
    pltpu.sync_copy(pos_ref, pos_v)
    iota = lax.iota(jnp.int32, LANES)
    sg = [(iota + dd * LANES) // 8 for dd in range(D // LANES)]
    sd = [(iota + dd * LANES) % 8 for dd in range(D // LANES)]
    zz = iota * 0

    def step(t, carry):
        half = t % 2
        sc = t // 2
        s0 = sc * SCH
        bb0 = (wid * BL + half * BH) // 16
        pltpu.sync_copy(x_ref.at[sc, pl.ds(bb0, 4)], idx_v)
        descs = [
            pltpu.async_copy(
                table_ref.at[idx_v.at[j]],
                buf_v.at[pl.ds(j * 128, 128)],
                sem,
            )
            for j in range(4)
        ]
        for dsc in descs:
            dsc.wait()

        def seq(sj, carry2):
            pvecs = [pos_v[s0 + sj, pl.ds(dd * LANES, LANES)]
                     for dd in range(D // LANES)]
            ob = obuf_v.at[sj]
            for b16 in range(BH // LANES):
                for bq in range(4):
                    rows = [b16 * 128 + sj * LANES + bq * 4 + bi
                            for bi in range(4)]
                    vals = [
                        buf_v[sj * 0 + r, pl.ds(dd * LANES, LANES)]
                        for r in rows
                        for dd in range(D // LANES)
                    ]
                    sums = [
                        vals[k * 4 + dd] + pvecs[dd]
                        for k in range(4)
                        for dd in range(D // LANES)
                    ]
                    for k in range(4):
                        bcol = b16 * 16 + bq * 4 + k
                        for dd in range(D // LANES):
                            plsc.store_scatter(
                                ob,
                                [sg[dd], zz, sd[dd], zz + bcol],
                                sums[k * 4 + dd],
                            )
            return carry2

        lax.fori_loop(0, SCH, seq, 0)
        pltpu.sync_copy(
            obuf_v.at[:, :, :, :, pl.ds(0, BH)],
            out_ref.at[
                pl.ds(s0, SCH), :, pl.ds(wid, 1), :, pl.ds(half * BH, BH)
            ],
        )
        return carry

    lax.fori_loop(0, NSTEP, step, 0)


@functools.lru_cache(maxsize=1)
def _make_gather_add():
    mesh = plsc.VectorSubcoreMesh(
        core_axis_name="c", subcore_axis_name="s", num_cores=NC, num_subcores=NS
    )
    return pl.kernel(
        _body,
        out_type=jax.ShapeDtypeStruct((S, DG, B // BL, 8, BL), jnp.float32),
        mesh=mesh,
        scratch_types=[
            pltpu.VMEM((4, 128), jnp.int32),
            pltpu.VMEM((BH * SCH, PITCH), jnp.float32),
            pltpu.VMEM((SCH, DG, 1, 8, BHP), jnp.float32),
            pltpu.VMEM((S, D), jnp.float32),
            pltpu.SemaphoreType.DMA,
        ],
        compiler_params=pltpu.CompilerParams(
            use_tc_tiling_on_sc=False, needs_layout_passes=False
        ),
    )


def kernel(x, table, pos_embed):
    x2 = x.astype(jnp.int32) * 2
    xb = (
        x2.reshape(B // 16, 16, S // SCH, SCH)
        .transpose(2, 0, 3, 1)
        .reshape(S // SCH, B // 16, 128)
    )
    table_w = jnp.pad(table, ((0, 0), (0, DP - D))).reshape(2 * VOCAB, D)
    pos2d = pos_embed.reshape(S, D).astype(jnp.float32)
    out5 = _make_gather_add()(xb, table_w, pos2d)
    return out5.transpose(2, 4, 0, 1, 3).reshape(B, S, D)
